# Initial kernel scaffold; baseline (speedup 1.0000x reference)
import jax
import jax.numpy as jnp
from jax.experimental import pallas as pl


def kernel(x, edge_index, batch, iaw_attr, W0, b0, W1, b1, W2, b2, iaw_W1, iaw_b1, iaw_W2, iaw_b2, ln_g, ln_b, mg_W1, mg_b1, mg_W2, mg_b2, p_W1, p_b1, p_W2, p_b2, p_W3, p_b3):
    raise NotImplementedError("write your pallas kernel here")


# speedup vs baseline: 5.6600x; 5.6600x over previous
import functools

import jax
import jax.numpy as jnp
from jax import lax
from jax.experimental import pallas as pl
from jax.experimental.pallas import tpu as pltpu
from jax.experimental.pallas import tpu_sc as plsc

_N = 10000
_E = 320000
_NG = 64
_DF = 128
_DH = 512
_DI = 256
_DM = 2 * _DH + _DI
_DP = 512
_NC = 2
_NS = 16
_CW = 128
_K = 80
_RB = 400
_GRID = _N // _RB
_RPT = _N // _NS


@functools.lru_cache(maxsize=None)
def _mesh():
    return plsc.VectorSubcoreMesh(core_axis_name="c", subcore_axis_name="s",
                                  num_cores=_NC, num_subcores=_NS)


def _zero_vmem_2d(ref, rows, cols):
    z16 = jnp.zeros((16,), jnp.float32)
    nc = cols // 16

    def body(i, _):
        r = i // nc
        q = (i % nc) * 16
        ref[r, pl.ds(q, 16)] = z16
        return 0

    lax.fori_loop(0, rows * nc, body, 0)


@functools.lru_cache(maxsize=None)
def _build_sc_degree():
    @functools.partial(
        pl.kernel,
        out_type=jax.ShapeDtypeStruct((_NC, _NS, _RPT, 16), jnp.float32),
        mesh=_mesh(),
        scratch_types=[
            pltpu.VMEM((_K,), jnp.int32),
            pltpu.VMEM((_K, 16), jnp.float32),
            pltpu.VMEM((125, 16), jnp.float32),
            pltpu.VMEM_SHARED((_N, 16), jnp.float32),
        ],
    )
    def sc_degree(dst_hbm, out_hbm, idx_v, ones_v, zrow_v, sdeg):
        c = lax.axis_index("c")
        s = lax.axis_index("s")
        wid = c * _NS + s

        one16 = jnp.full((16,), 1.0, jnp.float32)

        def fill(i, _):
            ones_v[i, :] = one16
            return 0

        lax.fori_loop(0, _K, fill, 0)
        _zero_vmem_2d(zrow_v, 125, 16)

        if True:
            def z(i, _):
                pltpu.sync_copy(zrow_v,
                                sdeg.at[pl.ds(s * _RPT + i * 125, 125)])
                return 0

            lax.fori_loop(0, _RPT // 125, z, 0)
            plsc.subcore_barrier()

            base = wid * (_E // (_NC * _NS))

            def step(i, _):
                pltpu.sync_copy(dst_hbm.at[pl.ds(base + i * _K, _K)], idx_v)
                pltpu.sync_copy(ones_v, sdeg.at[idx_v], add=True)
                return 0

            lax.fori_loop(0, (_E // (_NC * _NS)) // _K, step, 0)
            plsc.subcore_barrier()
            pltpu.sync_copy(
                sdeg.at[pl.ds(s * _RPT, _RPT)],
                out_hbm.at[c, s],
            )

    return sc_degree


@functools.lru_cache(maxsize=None)
def _build_sc_conv():
    @functools.partial(
        pl.kernel,
        out_type=[jax.ShapeDtypeStruct((_NS, _RPT, _CW), jnp.float32)] * 4,
        mesh=_mesh(),
        scratch_types=[
            pltpu.VMEM((_K,), jnp.int32),
            pltpu.VMEM((_K,), jnp.int32),
            pltpu.VMEM((_K, _CW), jnp.float32),
            pltpu.VMEM((125, _CW), jnp.float32),
            pltpu.SemaphoreType.DMA,
            pltpu.VMEM_SHARED((_N, _CW), jnp.float32),
        ],
    )
    def sc_conv(h0, h1, h2, h3, src_hbm, dst_hbm, o0, o1, o2, o3,
                sidx, didx, rbuf, zrow, sem, acc):
        c = lax.axis_index("c")
        s = lax.axis_index("s")
        _zero_vmem_2d(zrow, 125, _CW)

        epw = _E // _NS

        if True:
            def one_chunk(hin, hout):
                def z(i, _):
                    pltpu.sync_copy(
                        zrow, acc.at[pl.ds(s * _RPT + i * 125, 125)])
                    return 0

                lax.fori_loop(0, _RPT // 125, z, 0)
                plsc.subcore_barrier()

                ebase = s * epw

                def step(i, _):
                    pltpu.sync_copy(
                        src_hbm.at[pl.ds(ebase + i * _K, _K)], sidx)
                    pltpu.sync_copy(
                        dst_hbm.at[pl.ds(ebase + i * _K, _K)], didx)
                    pltpu.async_copy(hin.at[sidx], rbuf, sem).wait()
                    pltpu.sync_copy(rbuf, acc.at[didx], add=True)
                    return 0

                lax.fori_loop(0, epw // _K, step, 0)
                plsc.subcore_barrier()
                pltpu.sync_copy(
                    acc.at[pl.ds(s * _RPT, _RPT)],
                    hout.at[s],
                )
                plsc.subcore_barrier()

            ins = [h0, h1, h2, h3]
            outs = [o0, o1, o2, o3]
            for ci in range(4):
                @pl.when(c == ci // 2)
                def _(ci=ci):
                    one_chunk(ins[ci], outs[ci])

    return sc_conv


@functools.lru_cache(maxsize=None)
def _build_sc_pool():
    @functools.partial(
        pl.kernel,
        out_type=[
            jax.ShapeDtypeStruct((_NC * _NS, _NG, _DH), jnp.float32)
        ] * 2,
        mesh=_mesh(),
        scratch_types=[
            pltpu.VMEM((64, 16), jnp.int32),
            pltpu.VMEM((64, _DH), jnp.float32),
            pltpu.VMEM((1, _DH), jnp.float32),
            pltpu.VMEM((1, _DH), jnp.float32),
            pltpu.VMEM((16,), jnp.int32),
            pltpu.VMEM((_NG, _DH), jnp.float32),
            pltpu.VMEM((_NG, _DH), jnp.float32),
        ],
    )
    def sc_pool(h_hbm, b16_hbm, omax_hbm, osum_hbm,
                bbuf, rbuf, am, asm, pv, pmax, psum):
        cc = lax.axis_index("c")
        s = lax.axis_index("s")
        wid = cc * _NS + s

        r0 = (wid * _N // (_NC * _NS)) // 16 * 16
        r1 = ((wid + 1) * _N // (_NC * _NS)) // 16 * 16

        neg = jnp.full((16,), -3.4e38, jnp.float32)
        z16 = jnp.zeros((16,), jnp.float32)
        nch = _DH // 16

        def initrow(i, _):
            r = i // nch
            q = (i % nch) * 16
            pmax[r, pl.ds(q, 16)] = neg
            psum[r, pl.ds(q, 16)] = z16
            return 0

        lax.fori_loop(0, _NG * nch, initrow, 0)
        pv[...] = jnp.full((16,), -1, jnp.int32)

        nblk = (r1 - r0 + 63) // 64

        def blk(i, _):
            u = r0 + i * 64
            b = jnp.minimum(u, _N - 64)
            pltpu.sync_copy(h_hbm.at[pl.ds(b, 64)], rbuf)
            pltpu.sync_copy(b16_hbm.at[pl.ds(b, 64)], bbuf)

            def row(r, _):
                gr = b + r

                @pl.when(jnp.logical_and(gr >= u, gr < r1))
                def _():
                    b16 = bbuf[r, :]
                    sg = b16[0]
                    prev = pv[...][0]
                    eqs = sg == prev
                    pv[...] = b16
                    for f in range(nch):
                        v = rbuf[r, pl.ds(f * 16, 16)]
                        m0 = am[0, pl.ds(f * 16, 16)]
                        s0 = asm[0, pl.ds(f * 16, 16)]
                        nm = jnp.where(eqs, jnp.maximum(m0, v), v)
                        ns = jnp.where(eqs, s0 + v, v)
                        am[0, pl.ds(f * 16, 16)] = nm
                        asm[0, pl.ds(f * 16, 16)] = ns
                        pmax[sg, pl.ds(f * 16, 16)] = nm
                        psum[sg, pl.ds(f * 16, 16)] = ns

                return 0

            lax.fori_loop(0, 64, row, 0)
            return 0

        lax.fori_loop(0, nblk, blk, 0)
        pltpu.sync_copy(pmax, omax_hbm.at[wid])
        pltpu.sync_copy(psum, osum_hbm.at[wid])

    return sc_pool


def _tc_pre(dp0, dp1, batch_col):
    def body(dp0_ref, dp1_ref, b_ref, dinv_ref, cf_ref, b16_ref, acc_ref):
        i = pl.program_id(0)
        deg = dp0_ref[:, 0:1] + dp1_ref[:, 0:1] + 1.0
        dinv_ref[...] = lax.rsqrt(deg)
        b16_ref[...] = jnp.broadcast_to(b_ref[...], (_RB, 16))

        oh = (lax.broadcasted_iota(jnp.int32, (_RB, _NG), 1)
              == jnp.broadcast_to(b_ref[...], (_RB, _NG))).astype(jnp.float32)
        part = jnp.dot(jnp.ones((1, _RB), jnp.float32), oh,
                       preferred_element_type=jnp.float32)

        @pl.when(i == 0)
        def _():
            acc_ref[...] = jnp.zeros((1, _NG), jnp.float32)

        acc_ref[...] += part

        @pl.when(i == _GRID - 1)
        def _():
            cr = acc_ref[...]
            i0 = lax.broadcasted_iota(jnp.int32, (_NG, _NG), 0)
            i1 = lax.broadcasted_iota(jnp.int32, (_NG, _NG), 1)
            eye = (i0 == i1).astype(jnp.float32)
            ones = jnp.ones((_NG, 1), jnp.float32)
            cf_ref[...] = jnp.dot(eye * cr, ones,
                                  preferred_element_type=jnp.float32)

    return pl.pallas_call(
        body,
        grid=(_GRID,),
        in_specs=[
            pl.BlockSpec((_RB, 16), lambda i: (i, 0)),
            pl.BlockSpec((_RB, 16), lambda i: (i, 0)),
            pl.BlockSpec((_RB, 1), lambda i: (i, 0)),
        ],
        scratch_shapes=[pltpu.VMEM((1, _NG), jnp.float32)],
        out_specs=[
            pl.BlockSpec((_RB, 1), lambda i: (i, 0)),
            pl.BlockSpec((_NG, 1), lambda i: (0, 0)),
            pl.BlockSpec((_RB, 16), lambda i: (i, 0)),
        ],
        out_shape=[
            jax.ShapeDtypeStruct((_N, 1), jnp.float32),
            jax.ShapeDtypeStruct((_NG, 1), jnp.float32),
            jax.ShapeDtypeStruct((_N, 16), jnp.int32),
        ],
    )(dp0, dp1, batch_col)


def _tc_mm_first(x, w, dinv):
    def body(x_ref, w_ref, d_ref, o0, o1, o2, o3):
        h = jnp.dot(x_ref[...], w_ref[...],
                    preferred_element_type=jnp.float32) * d_ref[...]
        for ci, o in enumerate((o0, o1, o2, o3)):
            o[...] = h[:, ci * _CW:(ci + 1) * _CW]

    return pl.pallas_call(
        body,
        grid=(_GRID,),
        in_specs=[
            pl.BlockSpec((_RB, _DF), lambda i: (i, 0)),
            pl.BlockSpec((_DF, _DH), lambda i: (0, 0)),
            pl.BlockSpec((_RB, 1), lambda i: (i, 0)),
        ],
        out_specs=[pl.BlockSpec((_RB, _CW), lambda i: (i, 0))] * 4,
        out_shape=[jax.ShapeDtypeStruct((_N, _CW), jnp.float32)] * 4,
    )(x, w, dinv)


def _tc_fused(sc, hws, dinv, b, w):
    def body(s0, s1, s2, s3, p0, p1, p2, p3, d_ref, b_ref, w_ref,
             o0, o1, o2, o3):
        d = d_ref[...]
        hs = []
        for ci, (sr, pr) in enumerate(zip((s0, s1, s2, s3),
                                          (p0, p1, p2, p3))):
            hs.append((sr[...] + pr[...]) * d
                      + b_ref[:, ci * _CW:(ci + 1) * _CW])
        h = jnp.concatenate(hs, axis=1)
        g = jnp.dot(h, w_ref[...], preferred_element_type=jnp.float32) * d
        for ci, o in enumerate((o0, o1, o2, o3)):
            o[...] = g[:, ci * _CW:(ci + 1) * _CW]

    return pl.pallas_call(
        body,
        grid=(_GRID,),
        in_specs=(
            [pl.BlockSpec((_RB, _CW), lambda i: (i, 0))] * 8
            + [pl.BlockSpec((_RB, 1), lambda i: (i, 0)),
               pl.BlockSpec((1, _DH), lambda i: (0, 0)),
               pl.BlockSpec((_DH, _DH), lambda i: (0, 0))]
        ),
        out_specs=[pl.BlockSpec((_RB, _CW), lambda i: (i, 0))] * 4,
        out_shape=[jax.ShapeDtypeStruct((_N, _CW), jnp.float32)] * 4,
    )(*sc, *hws, dinv, b, w)


def _tc_final(sc, hws, dinv, b):
    def body(s0, s1, s2, s3, p0, p1, p2, p3, d_ref, b_ref, o_ref):
        d = d_ref[...]
        hs = []
        for ci, (sr, pr) in enumerate(zip((s0, s1, s2, s3),
                                          (p0, p1, p2, p3))):
            hs.append((sr[...] + pr[...]) * d
                      + b_ref[:, ci * _CW:(ci + 1) * _CW])
        o_ref[...] = jnp.concatenate(hs, axis=1)

    return pl.pallas_call(
        body,
        grid=(_GRID,),
        in_specs=(
            [pl.BlockSpec((_RB, _CW), lambda i: (i, 0))] * 8
            + [pl.BlockSpec((_RB, 1), lambda i: (i, 0)),
               pl.BlockSpec((1, _DH), lambda i: (0, 0))]
        ),
        out_specs=pl.BlockSpec((_RB, _DH), lambda i: (i, 0)),
        out_shape=jax.ShapeDtypeStruct((_N, _DH), jnp.float32),
    )(*sc, *hws, dinv, b)


def _tc_head(pmax, psum, cnt_f, iaw_attr, iW1, ib1, iW2, ib2, lng, lnb,
             mW1, mb1, mW2, mb2, pW1, pb1, pW2, pb2, pW3, pb3):
    def body(pmax_ref, psum_ref, cnt_ref, attr_ref, iW1_ref, ib1_ref,
             iW2_ref, ib2_ref, lng_ref, lnb_ref, mW1_ref, mb1_ref,
             mW2_ref, mb2_ref, pW1_ref, pb1_ref, pW2_ref, pb2_ref,
             pW3_ref, pb3_ref, out_ref):
        def leaky(t):
            return jnp.where(t >= 0, t, 0.2 * t)

        def elu(t):
            return jnp.where(t > 0, t, jnp.exp(jnp.minimum(t, 0.0)) - 1.0)

        xm = jnp.max(pmax_ref[...], axis=0)
        xsum = jnp.sum(psum_ref[...], axis=0)
        xmean = xsum / jnp.maximum(cnt_ref[...], 1.0)

        iaw = jnp.dot(attr_ref[...], iW1_ref[...],
                      preferred_element_type=jnp.float32) + ib1_ref[...]
        iaw = jnp.dot(leaky(iaw), iW2_ref[...],
                      preferred_element_type=jnp.float32) + ib2_ref[...]
        mu = jnp.mean(iaw, axis=1, keepdims=True)
        var = jnp.mean((iaw - mu) * (iaw - mu), axis=1, keepdims=True)
        iawn = (iaw - mu) * lax.rsqrt(var + 1e-5) * lng_ref[...] + lnb_ref[...]

        m = (jnp.dot(xm, mW1_ref[0:_DH, :],
                     preferred_element_type=jnp.float32)
             + jnp.dot(xmean, mW1_ref[_DH:2 * _DH, :],
                       preferred_element_type=jnp.float32)
             + jnp.dot(iawn, mW1_ref[2 * _DH:_DM, :],
                       preferred_element_type=jnp.float32)
             + mb1_ref[...])
        m = jnp.dot(leaky(m), mW2_ref[...],
                    preferred_element_type=jnp.float32) + mb2_ref[...]
        h2 = elu(jnp.dot(m, pW1_ref[...],
                         preferred_element_type=jnp.float32) + pb1_ref[...])
        h2 = elu(jnp.dot(h2, pW2_ref[...],
                         preferred_element_type=jnp.float32) + pb2_ref[...])
        out_ref[...] = jnp.dot(h2, pW3_ref[...],
                               preferred_element_type=jnp.float32) + pb3_ref[...]

    return pl.pallas_call(
        body,
        out_shape=jax.ShapeDtypeStruct((_NG, 1), jnp.float32),
    )(pmax, psum, cnt_f, iaw_attr, iW1, ib1, iW2, ib2, lng, lnb,
      mW1, mb1, mW2, mb2, pW1, pb1, pW2, pb2, pW3, pb3)


def kernel(x, edge_index, batch, iaw_attr, W0, b0, W1, b1, W2, b2,
           iaw_W1, iaw_b1, iaw_W2, iaw_b2, ln_g, ln_b,
           mg_W1, mg_b1, mg_W2, mg_b2, p_W1, p_b1, p_W2, p_b2, p_W3, p_b3):
    src = edge_index[0]
    dst = edge_index[1]

    dp = _build_sc_degree()(dst)
    dinv, cnt_f, batch16 = _tc_pre(dp[0].reshape(_N, 16),
                                   dp[1].reshape(_N, 16),
                                   batch.reshape(_N, 1))

    sc_conv = _build_sc_conv()

    def conv(hws_chunks):
        outs = sc_conv(*hws_chunks, src, dst)
        return [o.reshape(_N, _CW) for o in outs]

    hws = _tc_mm_first(x, W0, dinv)
    s1 = conv(hws)
    hws2 = _tc_fused(s1, hws, dinv, b0.reshape(1, _DH), W1)
    s2 = conv(hws2)
    hws3 = _tc_fused(s2, hws2, dinv, b1.reshape(1, _DH), W2)
    s3 = conv(hws3)
    h3 = _tc_final(s3, hws3, dinv, b2.reshape(1, _DH))

    pmax, psum = _build_sc_pool()(h3, batch16)

    return _tc_head(pmax, psum, cnt_f, iaw_attr,
                    iaw_W1, iaw_b1.reshape(1, _DI),
                    iaw_W2, iaw_b2.reshape(1, _DI),
                    ln_g.reshape(1, _DI), ln_b.reshape(1, _DI),
                    mg_W1, mg_b1.reshape(1, _DM), mg_W2, mg_b2.reshape(1, _DM),
                    p_W1, p_b1.reshape(1, _DP), p_W2, p_b2.reshape(1, _DP),
                    p_W3, p_b3.reshape(1, 1))


# speedup vs baseline: 11.3946x; 2.0132x over previous
import functools

import jax
import jax.numpy as jnp
from jax import lax
from jax.experimental import pallas as pl
from jax.experimental.pallas import tpu as pltpu
from jax.experimental.pallas import tpu_sc as plsc

_N = 10000
_E = 320000
_NG = 64
_DF = 128
_DH = 512
_DI = 256
_DM = 2 * _DH + _DI
_DP = 512
_NC = 2
_NS = 16
_CW = 128
_K = 80
_RB = 400
_GRID = _N // _RB
_RPT = _N // _NS


@functools.lru_cache(maxsize=None)
def _mesh():
    return plsc.VectorSubcoreMesh(core_axis_name="c", subcore_axis_name="s",
                                  num_cores=_NC, num_subcores=_NS)


def _zero_vmem_2d(ref, rows, cols):
    z16 = jnp.zeros((16,), jnp.float32)
    nc = cols // 16

    def body(i, _):
        r = i // nc
        q = (i % nc) * 16
        ref[r, pl.ds(q, 16)] = z16
        return 0

    lax.fori_loop(0, rows * nc, body, 0)


@functools.lru_cache(maxsize=None)
def _build_sc_degree():
    @functools.partial(
        pl.kernel,
        out_type=jax.ShapeDtypeStruct((_NC, _NS, _RPT, 16), jnp.float32),
        mesh=_mesh(),
        scratch_types=[
            pltpu.VMEM((_K,), jnp.int32),
            pltpu.VMEM((_K, 16), jnp.float32),
            pltpu.VMEM((125, 16), jnp.float32),
            pltpu.VMEM_SHARED((_N, 16), jnp.float32),
        ],
    )
    def sc_degree(dst_hbm, out_hbm, idx_v, ones_v, zrow_v, sdeg):
        c = lax.axis_index("c")
        s = lax.axis_index("s")
        wid = c * _NS + s

        one16 = jnp.full((16,), 1.0, jnp.float32)

        def fill(i, _):
            ones_v[i, :] = one16
            return 0

        lax.fori_loop(0, _K, fill, 0)
        _zero_vmem_2d(zrow_v, 125, 16)

        if True:
            def z(i, _):
                pltpu.sync_copy(zrow_v,
                                sdeg.at[pl.ds(s * _RPT + i * 125, 125)])
                return 0

            lax.fori_loop(0, _RPT // 125, z, 0)
            plsc.subcore_barrier()

            base = wid * (_E // (_NC * _NS))

            def step(i, _):
                pltpu.sync_copy(dst_hbm.at[pl.ds(base + i * _K, _K)], idx_v)
                pltpu.sync_copy(ones_v, sdeg.at[idx_v], add=True)
                return 0

            lax.fori_loop(0, (_E // (_NC * _NS)) // _K, step, 0)
            plsc.subcore_barrier()
            pltpu.sync_copy(
                sdeg.at[pl.ds(s * _RPT, _RPT)],
                out_hbm.at[c, s],
            )

    return sc_degree


_KC = 128
_EPT = 20096
_EP = _NS * _EPT
_NIT = _EPT // _KC
_NPAD = _N + 8


@functools.lru_cache(maxsize=None)
def _build_sc_conv():
    @functools.partial(
        pl.kernel,
        out_type=[jax.ShapeDtypeStruct((_NS, _RPT, _CW), jnp.float32)] * 4,
        mesh=_mesh(),
        scratch_types=[
            [pltpu.VMEM((_KC,), jnp.int32) for _ in range(4)],
            [pltpu.VMEM((_KC,), jnp.int32) for _ in range(4)],
            [pltpu.VMEM((_KC, _CW), jnp.float32) for _ in range(2)],
            pltpu.VMEM((25, _CW), jnp.float32),
            [pltpu.SemaphoreType.DMA for _ in range(4)],
            [pltpu.SemaphoreType.DMA for _ in range(2)],
            [pltpu.SemaphoreType.DMA for _ in range(2)],
            pltpu.VMEM_SHARED((_NPAD, _CW), jnp.float32),
        ],
    )
    def sc_conv(h0, h1, h2, h3, src_hbm, dst_hbm, o0, o1, o2, o3,
                sidx, didx, rbufs, zrow, isems, gsems, ssems, acc):
        c = lax.axis_index("c")
        s = lax.axis_index("s")
        _zero_vmem_2d(zrow, 25, _CW)
        ebase = s * _EPT
        n = _NIT

        def idx_load(j, sl):
            pltpu.async_copy(
                src_hbm.at[pl.ds(ebase + j * _KC, _KC)], sidx[sl],
                isems[sl])
            pltpu.async_copy(
                dst_hbm.at[pl.ds(ebase + j * _KC, _KC)], didx[sl],
                isems[sl])

        def idx_wait(j, sl):
            pltpu.make_async_copy(
                src_hbm.at[pl.ds(ebase + j * _KC, _KC)], sidx[sl],
                isems[sl]).wait()
            pltpu.make_async_copy(
                dst_hbm.at[pl.ds(ebase + j * _KC, _KC)], didx[sl],
                isems[sl]).wait()

        def one_chunk(hin, hout):
            def z(i, _):
                pltpu.sync_copy(
                    zrow, acc.at[pl.ds(s * _RPT + i * 25, 25)])
                return 0

            lax.fori_loop(0, _RPT // 25, z, 0)
            plsc.subcore_barrier()

            for p01 in range(2):
                idx_load(p01, p01)

            def step4(t, _):
                for p in range(4):
                    j = t * 4 + p

                    @pl.when(jnp.logical_and(j >= 2, j - 2 < n))
                    def _(j=j, p=p):
                        pltpu.make_async_copy(
                            rbufs[p % 2], acc.at[didx[(p + 2) % 4]],
                            ssems[p % 2]).wait()

                    @pl.when(j + 2 < n)
                    def _(j=j, p=p):
                        idx_load(j + 2, (p + 2) % 4)

                    @pl.when(j < n)
                    def _(j=j, p=p):
                        idx_wait(j, p % 4)
                        pltpu.async_copy(
                            hin.at[sidx[p % 4]], rbufs[p % 2],
                            gsems[p % 2])

                    jj = j - 1
                    q = (p + 3) % 4

                    @pl.when(jnp.logical_and(jj >= 0, jj < n))
                    def _(jj=jj, q=q):
                        pltpu.make_async_copy(
                            hin.at[sidx[q]], rbufs[q % 2],
                            gsems[q % 2]).wait()
                        pltpu.async_copy(
                            rbufs[q % 2], acc.at[didx[q]],
                            ssems[q % 2], add=True)

                return 0

            lax.fori_loop(0, (n + 1 + 4) // 4 + 1, step4, 0)
            plsc.subcore_barrier()
            pltpu.sync_copy(
                acc.at[pl.ds(s * _RPT, _RPT)],
                hout.at[s],
            )
            plsc.subcore_barrier()

        ins = [h0, h1, h2, h3]
        outs = [o0, o1, o2, o3]
        for ci in range(4):
            @pl.when(c == ci // 2)
            def _(ci=ci):
                one_chunk(ins[ci], outs[ci])

    return sc_conv


@functools.lru_cache(maxsize=None)
def _build_sc_pool():
    @functools.partial(
        pl.kernel,
        out_type=[
            jax.ShapeDtypeStruct((_NC * _NS, _NG, _DH), jnp.float32)
        ] * 2,
        mesh=_mesh(),
        scratch_types=[
            pltpu.VMEM((64, 16), jnp.int32),
            pltpu.VMEM((64, _DH), jnp.float32),
            pltpu.VMEM((1, _DH), jnp.float32),
            pltpu.VMEM((1, _DH), jnp.float32),
            pltpu.VMEM((16,), jnp.int32),
            pltpu.VMEM((_NG, _DH), jnp.float32),
            pltpu.VMEM((_NG, _DH), jnp.float32),
        ],
    )
    def sc_pool(h_hbm, b16_hbm, omax_hbm, osum_hbm,
                bbuf, rbuf, am, asm, pv, pmax, psum):
        cc = lax.axis_index("c")
        s = lax.axis_index("s")
        wid = cc * _NS + s

        r0 = (wid * _N // (_NC * _NS)) // 16 * 16
        r1 = ((wid + 1) * _N // (_NC * _NS)) // 16 * 16

        neg = jnp.full((16,), -3.4e38, jnp.float32)
        z16 = jnp.zeros((16,), jnp.float32)
        nch = _DH // 16

        def initrow(i, _):
            r = i // nch
            q = (i % nch) * 16
            pmax[r, pl.ds(q, 16)] = neg
            psum[r, pl.ds(q, 16)] = z16
            return 0

        lax.fori_loop(0, _NG * nch, initrow, 0)
        pv[...] = jnp.full((16,), -1, jnp.int32)

        nblk = (r1 - r0 + 63) // 64

        def blk(i, _):
            u = r0 + i * 64
            b = jnp.minimum(u, _N - 64)
            pltpu.sync_copy(h_hbm.at[pl.ds(b, 64)], rbuf)
            pltpu.sync_copy(b16_hbm.at[pl.ds(b, 64)], bbuf)

            def row(r, _):
                gr = b + r

                @pl.when(jnp.logical_and(gr >= u, gr < r1))
                def _():
                    b16 = bbuf[r, :]
                    sg = b16[0]
                    prev = pv[...][0]
                    eqs = sg == prev
                    pv[...] = b16
                    for f in range(nch):
                        v = rbuf[r, pl.ds(f * 16, 16)]
                        m0 = am[0, pl.ds(f * 16, 16)]
                        s0 = asm[0, pl.ds(f * 16, 16)]
                        nm = jnp.where(eqs, jnp.maximum(m0, v), v)
                        ns = jnp.where(eqs, s0 + v, v)
                        am[0, pl.ds(f * 16, 16)] = nm
                        asm[0, pl.ds(f * 16, 16)] = ns
                        pmax[sg, pl.ds(f * 16, 16)] = nm
                        psum[sg, pl.ds(f * 16, 16)] = ns

                return 0

            lax.fori_loop(0, 64, row, 0)
            return 0

        lax.fori_loop(0, nblk, blk, 0)
        pltpu.sync_copy(pmax, omax_hbm.at[wid])
        pltpu.sync_copy(psum, osum_hbm.at[wid])

    return sc_pool


def _tc_pre(dp0, dp1, batch_col):
    def body(dp0_ref, dp1_ref, b_ref, dinv_ref, cf_ref, b16_ref, acc_ref):
        i = pl.program_id(0)
        deg = dp0_ref[:, 0:1] + dp1_ref[:, 0:1] + 1.0
        dinv_ref[...] = lax.rsqrt(deg)
        b16_ref[...] = jnp.broadcast_to(b_ref[...], (_RB, 16))

        oh = (lax.broadcasted_iota(jnp.int32, (_RB, _NG), 1)
              == jnp.broadcast_to(b_ref[...], (_RB, _NG))).astype(jnp.float32)
        part = jnp.dot(jnp.ones((1, _RB), jnp.float32), oh,
                       preferred_element_type=jnp.float32)

        @pl.when(i == 0)
        def _():
            acc_ref[...] = jnp.zeros((1, _NG), jnp.float32)

        acc_ref[...] += part

        @pl.when(i == _GRID - 1)
        def _():
            cr = acc_ref[...]
            i0 = lax.broadcasted_iota(jnp.int32, (_NG, _NG), 0)
            i1 = lax.broadcasted_iota(jnp.int32, (_NG, _NG), 1)
            eye = (i0 == i1).astype(jnp.float32)
            ones = jnp.ones((_NG, 1), jnp.float32)
            cf_ref[...] = jnp.dot(eye * cr, ones,
                                  preferred_element_type=jnp.float32)

    return pl.pallas_call(
        body,
        grid=(_GRID,),
        in_specs=[
            pl.BlockSpec((_RB, 16), lambda i: (i, 0)),
            pl.BlockSpec((_RB, 16), lambda i: (i, 0)),
            pl.BlockSpec((_RB, 1), lambda i: (i, 0)),
        ],
        scratch_shapes=[pltpu.VMEM((1, _NG), jnp.float32)],
        out_specs=[
            pl.BlockSpec((_RB, 1), lambda i: (i, 0)),
            pl.BlockSpec((_NG, 1), lambda i: (0, 0)),
            pl.BlockSpec((_RB, 16), lambda i: (i, 0)),
        ],
        out_shape=[
            jax.ShapeDtypeStruct((_N, 1), jnp.float32),
            jax.ShapeDtypeStruct((_NG, 1), jnp.float32),
            jax.ShapeDtypeStruct((_N, 16), jnp.int32),
        ],
    )(dp0, dp1, batch_col)


def _tc_mm_first(x, w, dinv):
    def body(x_ref, w_ref, d_ref, o0, o1, o2, o3):
        h = jnp.dot(x_ref[...], w_ref[...],
                    preferred_element_type=jnp.float32) * d_ref[...]
        for ci, o in enumerate((o0, o1, o2, o3)):
            o[...] = h[:, ci * _CW:(ci + 1) * _CW]

    return pl.pallas_call(
        body,
        grid=(_GRID,),
        in_specs=[
            pl.BlockSpec((_RB, _DF), lambda i: (i, 0)),
            pl.BlockSpec((_DF, _DH), lambda i: (0, 0)),
            pl.BlockSpec((_RB, 1), lambda i: (i, 0)),
        ],
        out_specs=[pl.BlockSpec((_RB, _CW), lambda i: (i, 0))] * 4,
        out_shape=[jax.ShapeDtypeStruct((_N, _CW), jnp.float32)] * 4,
    )(x, w, dinv)


def _tc_fused(sc, hws, dinv, b, w):
    def body(s0, s1, s2, s3, p0, p1, p2, p3, d_ref, b_ref, w_ref,
             o0, o1, o2, o3):
        d = d_ref[...]
        hs = []
        for ci, (sr, pr) in enumerate(zip((s0, s1, s2, s3),
                                          (p0, p1, p2, p3))):
            hs.append((sr[...] + pr[...]) * d
                      + b_ref[:, ci * _CW:(ci + 1) * _CW])
        h = jnp.concatenate(hs, axis=1)
        g = jnp.dot(h, w_ref[...], preferred_element_type=jnp.float32) * d
        for ci, o in enumerate((o0, o1, o2, o3)):
            o[...] = g[:, ci * _CW:(ci + 1) * _CW]

    return pl.pallas_call(
        body,
        grid=(_GRID,),
        in_specs=(
            [pl.BlockSpec((_RB, _CW), lambda i: (i, 0))] * 8
            + [pl.BlockSpec((_RB, 1), lambda i: (i, 0)),
               pl.BlockSpec((1, _DH), lambda i: (0, 0)),
               pl.BlockSpec((_DH, _DH), lambda i: (0, 0))]
        ),
        out_specs=[pl.BlockSpec((_RB, _CW), lambda i: (i, 0))] * 4,
        out_shape=[jax.ShapeDtypeStruct((_N, _CW), jnp.float32)] * 4,
    )(*sc, *hws, dinv, b, w)


def _tc_final(sc, hws, dinv, b):
    def body(s0, s1, s2, s3, p0, p1, p2, p3, d_ref, b_ref, o_ref):
        d = d_ref[...]
        hs = []
        for ci, (sr, pr) in enumerate(zip((s0, s1, s2, s3),
                                          (p0, p1, p2, p3))):
            hs.append((sr[...] + pr[...]) * d
                      + b_ref[:, ci * _CW:(ci + 1) * _CW])
        o_ref[...] = jnp.concatenate(hs, axis=1)

    return pl.pallas_call(
        body,
        grid=(_GRID,),
        in_specs=(
            [pl.BlockSpec((_RB, _CW), lambda i: (i, 0))] * 8
            + [pl.BlockSpec((_RB, 1), lambda i: (i, 0)),
               pl.BlockSpec((1, _DH), lambda i: (0, 0))]
        ),
        out_specs=pl.BlockSpec((_RB, _DH), lambda i: (i, 0)),
        out_shape=jax.ShapeDtypeStruct((_N, _DH), jnp.float32),
    )(*sc, *hws, dinv, b)


def _tc_head(pmax, psum, cnt_f, iaw_attr, iW1, ib1, iW2, ib2, lng, lnb,
             mW1, mb1, mW2, mb2, pW1, pb1, pW2, pb2, pW3, pb3):
    def body(pmax_ref, psum_ref, cnt_ref, attr_ref, iW1_ref, ib1_ref,
             iW2_ref, ib2_ref, lng_ref, lnb_ref, mW1_ref, mb1_ref,
             mW2_ref, mb2_ref, pW1_ref, pb1_ref, pW2_ref, pb2_ref,
             pW3_ref, pb3_ref, out_ref):
        def leaky(t):
            return jnp.where(t >= 0, t, 0.2 * t)

        def elu(t):
            return jnp.where(t > 0, t, jnp.exp(jnp.minimum(t, 0.0)) - 1.0)

        xm = jnp.max(pmax_ref[...], axis=0)
        xsum = jnp.sum(psum_ref[...], axis=0)
        xmean = xsum / jnp.maximum(cnt_ref[...], 1.0)

        iaw = jnp.dot(attr_ref[...], iW1_ref[...],
                      preferred_element_type=jnp.float32) + ib1_ref[...]
        iaw = jnp.dot(leaky(iaw), iW2_ref[...],
                      preferred_element_type=jnp.float32) + ib2_ref[...]
        mu = jnp.mean(iaw, axis=1, keepdims=True)
        var = jnp.mean((iaw - mu) * (iaw - mu), axis=1, keepdims=True)
        iawn = (iaw - mu) * lax.rsqrt(var + 1e-5) * lng_ref[...] + lnb_ref[...]

        m = (jnp.dot(xm, mW1_ref[0:_DH, :],
                     preferred_element_type=jnp.float32)
             + jnp.dot(xmean, mW1_ref[_DH:2 * _DH, :],
                       preferred_element_type=jnp.float32)
             + jnp.dot(iawn, mW1_ref[2 * _DH:_DM, :],
                       preferred_element_type=jnp.float32)
             + mb1_ref[...])
        m = jnp.dot(leaky(m), mW2_ref[...],
                    preferred_element_type=jnp.float32) + mb2_ref[...]
        h2 = elu(jnp.dot(m, pW1_ref[...],
                         preferred_element_type=jnp.float32) + pb1_ref[...])
        h2 = elu(jnp.dot(h2, pW2_ref[...],
                         preferred_element_type=jnp.float32) + pb2_ref[...])
        out_ref[...] = jnp.dot(h2, pW3_ref[...],
                               preferred_element_type=jnp.float32) + pb3_ref[...]

    return pl.pallas_call(
        body,
        out_shape=jax.ShapeDtypeStruct((_NG, 1), jnp.float32),
    )(pmax, psum, cnt_f, iaw_attr, iW1, ib1, iW2, ib2, lng, lnb,
      mW1, mb1, mW2, mb2, pW1, pb1, pW2, pb2, pW3, pb3)


def kernel(x, edge_index, batch, iaw_attr, W0, b0, W1, b1, W2, b2,
           iaw_W1, iaw_b1, iaw_W2, iaw_b2, ln_g, ln_b,
           mg_W1, mg_b1, mg_W2, mg_b2, p_W1, p_b1, p_W2, p_b2, p_W3, p_b3):
    src = edge_index[0]
    dst = edge_index[1]

    dp = _build_sc_degree()(dst)
    dinv, cnt_f, batch16 = _tc_pre(dp[0].reshape(_N, 16),
                                   dp[1].reshape(_N, 16),
                                   batch.reshape(_N, 1))

    sc_conv = _build_sc_conv()
    src1 = jnp.concatenate([src, jnp.zeros((_EP - _E,), jnp.int32)])
    dst1 = jnp.concatenate([dst, jnp.full((_EP - _E,), _N, jnp.int32)])

    def conv(hws_chunks):
        outs = sc_conv(*hws_chunks, src1, dst1)
        return [o.reshape(_N, _CW) for o in outs]

    hws = _tc_mm_first(x, W0, dinv)
    s1 = conv(hws)
    hws2 = _tc_fused(s1, hws, dinv, b0.reshape(1, _DH), W1)
    s2 = conv(hws2)
    hws3 = _tc_fused(s2, hws2, dinv, b1.reshape(1, _DH), W2)
    s3 = conv(hws3)
    h3 = _tc_final(s3, hws3, dinv, b2.reshape(1, _DH))

    pmax, psum = _build_sc_pool()(h3, batch16)

    return _tc_head(pmax, psum, cnt_f, iaw_attr,
                    iaw_W1, iaw_b1.reshape(1, _DI),
                    iaw_W2, iaw_b2.reshape(1, _DI),
                    ln_g.reshape(1, _DI), ln_b.reshape(1, _DI),
                    mg_W1, mg_b1.reshape(1, _DM), mg_W2, mg_b2.reshape(1, _DM),
                    p_W1, p_b1.reshape(1, _DP), p_W2, p_b2.reshape(1, _DP),
                    p_W3, p_b3.reshape(1, 1))


# speedup vs baseline: 13.1151x; 1.1510x over previous
import functools

import jax
import jax.numpy as jnp
from jax import lax
from jax.experimental import pallas as pl
from jax.experimental.pallas import tpu as pltpu
from jax.experimental.pallas import tpu_sc as plsc

_N = 10000
_E = 320000
_NG = 64
_DF = 128
_DH = 512
_DI = 256
_DM = 2 * _DH + _DI
_DP = 512
_NC = 2
_NS = 16
_CW = 128
_K = 80
_RB = 400
_GRID = _N // _RB
_RPT = _N // _NS


@functools.lru_cache(maxsize=None)
def _mesh():
    return plsc.VectorSubcoreMesh(core_axis_name="c", subcore_axis_name="s",
                                  num_cores=_NC, num_subcores=_NS)


def _zero_vmem_2d(ref, rows, cols):
    z16 = jnp.zeros((16,), jnp.float32)
    nc = cols // 16

    def body(i, _):
        r = i // nc
        q = (i % nc) * 16
        ref[r, pl.ds(q, 16)] = z16
        return 0

    lax.fori_loop(0, rows * nc, body, 0)


@functools.lru_cache(maxsize=None)
def _build_sc_degree():
    @functools.partial(
        pl.kernel,
        out_type=jax.ShapeDtypeStruct((_NC, _NS, _RPT, 16), jnp.float32),
        mesh=_mesh(),
        scratch_types=[
            pltpu.VMEM((_K,), jnp.int32),
            pltpu.VMEM((_K, 16), jnp.float32),
            pltpu.VMEM((125, 16), jnp.float32),
            pltpu.VMEM_SHARED((_N, 16), jnp.float32),
        ],
    )
    def sc_degree(dst_hbm, out_hbm, idx_v, ones_v, zrow_v, sdeg):
        c = lax.axis_index("c")
        s = lax.axis_index("s")
        wid = c * _NS + s

        one16 = jnp.full((16,), 1.0, jnp.float32)

        def fill(i, _):
            ones_v[i, :] = one16
            return 0

        lax.fori_loop(0, _K, fill, 0)
        _zero_vmem_2d(zrow_v, 125, 16)

        if True:
            def z(i, _):
                pltpu.sync_copy(zrow_v,
                                sdeg.at[pl.ds(s * _RPT + i * 125, 125)])
                return 0

            lax.fori_loop(0, _RPT // 125, z, 0)
            plsc.subcore_barrier()

            base = wid * (_E // (_NC * _NS))

            def step(i, _):
                pltpu.sync_copy(dst_hbm.at[pl.ds(base + i * _K, _K)], idx_v)
                pltpu.sync_copy(ones_v, sdeg.at[idx_v], add=True)
                return 0

            lax.fori_loop(0, (_E // (_NC * _NS)) // _K, step, 0)
            plsc.subcore_barrier()
            pltpu.sync_copy(
                sdeg.at[pl.ds(s * _RPT, _RPT)],
                out_hbm.at[c, s],
            )

    return sc_degree


_KC = 96
_NIT = 209
_EPT = _KC * _NIT
_EP = _NS * _EPT
_NPAD = _N + 8


@functools.lru_cache(maxsize=None)
def _build_sc_conv():
    @functools.partial(
        pl.kernel,
        out_type=[jax.ShapeDtypeStruct((_NS, _RPT, _CW), jnp.float32)] * 4,
        mesh=_mesh(),
        scratch_types=[
            [pltpu.VMEM((_KC,), jnp.int32) for _ in range(6)],
            [pltpu.VMEM((_KC,), jnp.int32) for _ in range(6)],
            [pltpu.VMEM((_KC, _CW), jnp.float32) for _ in range(3)],
            pltpu.VMEM((25, _CW), jnp.float32),
            [pltpu.SemaphoreType.DMA for _ in range(6)],
            [pltpu.SemaphoreType.DMA for _ in range(3)],
            [pltpu.SemaphoreType.DMA for _ in range(3)],
            pltpu.VMEM_SHARED((_NPAD, _CW), jnp.float32),
        ],
    )
    def sc_conv(h0, h1, h2, h3, src_hbm, dst_hbm, o0, o1, o2, o3,
                sidx, didx, rbufs, zrow, isems, gsems, ssems, acc):
        c = lax.axis_index("c")
        s = lax.axis_index("s")
        _zero_vmem_2d(zrow, 25, _CW)
        ebase = s * _EPT
        n = _NIT

        def idx_load(j, sl):
            pltpu.async_copy(
                src_hbm.at[pl.ds(ebase + j * _KC, _KC)], sidx[sl],
                isems[sl])
            pltpu.async_copy(
                dst_hbm.at[pl.ds(ebase + j * _KC, _KC)], didx[sl],
                isems[sl])

        def idx_wait(j, sl):
            pltpu.make_async_copy(
                src_hbm.at[pl.ds(ebase + j * _KC, _KC)], sidx[sl],
                isems[sl]).wait()
            pltpu.make_async_copy(
                dst_hbm.at[pl.ds(ebase + j * _KC, _KC)], didx[sl],
                isems[sl]).wait()

        def one_chunk(hin, hout):
            def z(i, _):
                pltpu.sync_copy(
                    zrow, acc.at[pl.ds(s * _RPT + i * 25, 25)])
                return 0

            lax.fori_loop(0, _RPT // 25, z, 0)
            plsc.subcore_barrier()

            for p01 in range(2):
                idx_load(p01, p01)

            def step6(t, _):
                for p in range(6):
                    j = t * 6 + p

                    @pl.when(jnp.logical_and(j >= 3, j - 3 < n))
                    def _(j=j, p=p):
                        pltpu.make_async_copy(
                            rbufs[p % 3], acc.at[didx[(p + 3) % 6]],
                            ssems[p % 3]).wait()

                    @pl.when(j + 2 < n)
                    def _(j=j, p=p):
                        idx_load(j + 2, (p + 2) % 6)

                    @pl.when(j < n)
                    def _(j=j, p=p):
                        idx_wait(j, p)
                        pltpu.async_copy(
                            hin.at[sidx[p]], rbufs[p % 3],
                            gsems[p % 3])

                    jj = j - 2
                    qi = (p + 4) % 6
                    qr = (p + 1) % 3

                    @pl.when(jnp.logical_and(jj >= 0, jj < n))
                    def _(jj=jj, qi=qi, qr=qr):
                        pltpu.make_async_copy(
                            hin.at[sidx[qi]], rbufs[qr],
                            gsems[qr]).wait()
                        pltpu.async_copy(
                            rbufs[qr], acc.at[didx[qi]],
                            ssems[qr], add=True)

                return 0

            lax.fori_loop(0, (n + 2 + 6) // 6 + 1, step6, 0)
            plsc.subcore_barrier()
            pltpu.sync_copy(
                acc.at[pl.ds(s * _RPT, _RPT)],
                hout.at[s],
            )
            plsc.subcore_barrier()

        ins = [h0, h1, h2, h3]
        outs = [o0, o1, o2, o3]
        for ci in range(4):
            @pl.when(c == ci // 2)
            def _(ci=ci):
                one_chunk(ins[ci], outs[ci])

    return sc_conv


@functools.lru_cache(maxsize=None)
def _build_sc_pool():
    @functools.partial(
        pl.kernel,
        out_type=[
            jax.ShapeDtypeStruct((_NC * _NS, _NG, _DH), jnp.float32)
        ] * 2,
        mesh=_mesh(),
        scratch_types=[
            pltpu.VMEM((64, 16), jnp.int32),
            pltpu.VMEM((64, _DH), jnp.float32),
            pltpu.VMEM((1, _DH), jnp.float32),
            pltpu.VMEM((1, _DH), jnp.float32),
            pltpu.VMEM((16,), jnp.int32),
            pltpu.VMEM((_NG, _DH), jnp.float32),
            pltpu.VMEM((_NG, _DH), jnp.float32),
        ],
    )
    def sc_pool(h_hbm, b16_hbm, omax_hbm, osum_hbm,
                bbuf, rbuf, am, asm, pv, pmax, psum):
        cc = lax.axis_index("c")
        s = lax.axis_index("s")
        wid = cc * _NS + s

        r0 = (wid * _N // (_NC * _NS)) // 16 * 16
        r1 = ((wid + 1) * _N // (_NC * _NS)) // 16 * 16

        neg = jnp.full((16,), -3.4e38, jnp.float32)
        z16 = jnp.zeros((16,), jnp.float32)
        nch = _DH // 16

        def initrow(i, _):
            r = i // nch
            q = (i % nch) * 16
            pmax[r, pl.ds(q, 16)] = neg
            psum[r, pl.ds(q, 16)] = z16
            return 0

        lax.fori_loop(0, _NG * nch, initrow, 0)
        pv[...] = jnp.full((16,), -1, jnp.int32)

        nblk = (r1 - r0 + 63) // 64

        def blk(i, _):
            u = r0 + i * 64
            b = jnp.minimum(u, _N - 64)
            pltpu.sync_copy(h_hbm.at[pl.ds(b, 64)], rbuf)
            pltpu.sync_copy(b16_hbm.at[pl.ds(b, 64)], bbuf)

            def row(r, _):
                gr = b + r

                @pl.when(jnp.logical_and(gr >= u, gr < r1))
                def _():
                    b16 = bbuf[r, :]
                    sg = b16[0]
                    prev = pv[...][0]
                    eqs = sg == prev
                    pv[...] = b16
                    for f in range(nch):
                        v = rbuf[r, pl.ds(f * 16, 16)]
                        m0 = am[0, pl.ds(f * 16, 16)]
                        s0 = asm[0, pl.ds(f * 16, 16)]
                        nm = jnp.where(eqs, jnp.maximum(m0, v), v)
                        ns = jnp.where(eqs, s0 + v, v)
                        am[0, pl.ds(f * 16, 16)] = nm
                        asm[0, pl.ds(f * 16, 16)] = ns
                        pmax[sg, pl.ds(f * 16, 16)] = nm
                        psum[sg, pl.ds(f * 16, 16)] = ns

                return 0

            lax.fori_loop(0, 64, row, 0)
            return 0

        lax.fori_loop(0, nblk, blk, 0)
        pltpu.sync_copy(pmax, omax_hbm.at[wid])
        pltpu.sync_copy(psum, osum_hbm.at[wid])

    return sc_pool


def _tc_pre(dp0, dp1, batch_col):
    def body(dp0_ref, dp1_ref, b_ref, dinv_ref, cf_ref, b16_ref, acc_ref):
        i = pl.program_id(0)
        deg = dp0_ref[:, 0:1] + dp1_ref[:, 0:1] + 1.0
        dinv_ref[...] = lax.rsqrt(deg)
        b16_ref[...] = jnp.broadcast_to(b_ref[...], (_RB, 16))

        oh = (lax.broadcasted_iota(jnp.int32, (_RB, _NG), 1)
              == jnp.broadcast_to(b_ref[...], (_RB, _NG))).astype(jnp.float32)
        part = jnp.dot(jnp.ones((1, _RB), jnp.float32), oh,
                       preferred_element_type=jnp.float32)

        @pl.when(i == 0)
        def _():
            acc_ref[...] = jnp.zeros((1, _NG), jnp.float32)

        acc_ref[...] += part

        @pl.when(i == _GRID - 1)
        def _():
            cr = acc_ref[...]
            i0 = lax.broadcasted_iota(jnp.int32, (_NG, _NG), 0)
            i1 = lax.broadcasted_iota(jnp.int32, (_NG, _NG), 1)
            eye = (i0 == i1).astype(jnp.float32)
            ones = jnp.ones((_NG, 1), jnp.float32)
            cf_ref[...] = jnp.dot(eye * cr, ones,
                                  preferred_element_type=jnp.float32)

    return pl.pallas_call(
        body,
        grid=(_GRID,),
        in_specs=[
            pl.BlockSpec((_RB, 16), lambda i: (i, 0)),
            pl.BlockSpec((_RB, 16), lambda i: (i, 0)),
            pl.BlockSpec((_RB, 1), lambda i: (i, 0)),
        ],
        scratch_shapes=[pltpu.VMEM((1, _NG), jnp.float32)],
        out_specs=[
            pl.BlockSpec((_RB, 1), lambda i: (i, 0)),
            pl.BlockSpec((_NG, 1), lambda i: (0, 0)),
            pl.BlockSpec((_RB, 16), lambda i: (i, 0)),
        ],
        out_shape=[
            jax.ShapeDtypeStruct((_N, 1), jnp.float32),
            jax.ShapeDtypeStruct((_NG, 1), jnp.float32),
            jax.ShapeDtypeStruct((_N, 16), jnp.int32),
        ],
    )(dp0, dp1, batch_col)


def _tc_mm_first(x, w, dinv):
    def body(x_ref, w_ref, d_ref, o0, o1, o2, o3):
        h = jnp.dot(x_ref[...], w_ref[...],
                    preferred_element_type=jnp.float32) * d_ref[...]
        for ci, o in enumerate((o0, o1, o2, o3)):
            o[...] = h[:, ci * _CW:(ci + 1) * _CW]

    return pl.pallas_call(
        body,
        grid=(_GRID,),
        in_specs=[
            pl.BlockSpec((_RB, _DF), lambda i: (i, 0)),
            pl.BlockSpec((_DF, _DH), lambda i: (0, 0)),
            pl.BlockSpec((_RB, 1), lambda i: (i, 0)),
        ],
        out_specs=[pl.BlockSpec((_RB, _CW), lambda i: (i, 0))] * 4,
        out_shape=[jax.ShapeDtypeStruct((_N, _CW), jnp.float32)] * 4,
    )(x, w, dinv)


def _tc_fused(sc, hws, dinv, b, w):
    def body(s0, s1, s2, s3, p0, p1, p2, p3, d_ref, b_ref, w_ref,
             o0, o1, o2, o3):
        d = d_ref[...]
        hs = []
        for ci, (sr, pr) in enumerate(zip((s0, s1, s2, s3),
                                          (p0, p1, p2, p3))):
            hs.append((sr[...] + pr[...]) * d
                      + b_ref[:, ci * _CW:(ci + 1) * _CW])
        h = jnp.concatenate(hs, axis=1)
        g = jnp.dot(h, w_ref[...], preferred_element_type=jnp.float32) * d
        for ci, o in enumerate((o0, o1, o2, o3)):
            o[...] = g[:, ci * _CW:(ci + 1) * _CW]

    return pl.pallas_call(
        body,
        grid=(_GRID,),
        in_specs=(
            [pl.BlockSpec((_RB, _CW), lambda i: (i, 0))] * 8
            + [pl.BlockSpec((_RB, 1), lambda i: (i, 0)),
               pl.BlockSpec((1, _DH), lambda i: (0, 0)),
               pl.BlockSpec((_DH, _DH), lambda i: (0, 0))]
        ),
        out_specs=[pl.BlockSpec((_RB, _CW), lambda i: (i, 0))] * 4,
        out_shape=[jax.ShapeDtypeStruct((_N, _CW), jnp.float32)] * 4,
    )(*sc, *hws, dinv, b, w)


def _tc_final(sc, hws, dinv, b):
    def body(s0, s1, s2, s3, p0, p1, p2, p3, d_ref, b_ref, o_ref):
        d = d_ref[...]
        hs = []
        for ci, (sr, pr) in enumerate(zip((s0, s1, s2, s3),
                                          (p0, p1, p2, p3))):
            hs.append((sr[...] + pr[...]) * d
                      + b_ref[:, ci * _CW:(ci + 1) * _CW])
        o_ref[...] = jnp.concatenate(hs, axis=1)

    return pl.pallas_call(
        body,
        grid=(_GRID,),
        in_specs=(
            [pl.BlockSpec((_RB, _CW), lambda i: (i, 0))] * 8
            + [pl.BlockSpec((_RB, 1), lambda i: (i, 0)),
               pl.BlockSpec((1, _DH), lambda i: (0, 0))]
        ),
        out_specs=pl.BlockSpec((_RB, _DH), lambda i: (i, 0)),
        out_shape=jax.ShapeDtypeStruct((_N, _DH), jnp.float32),
    )(*sc, *hws, dinv, b)


def _tc_head(pmax, psum, cnt_f, iaw_attr, iW1, ib1, iW2, ib2, lng, lnb,
             mW1, mb1, mW2, mb2, pW1, pb1, pW2, pb2, pW3, pb3):
    def body(pmax_ref, psum_ref, cnt_ref, attr_ref, iW1_ref, ib1_ref,
             iW2_ref, ib2_ref, lng_ref, lnb_ref, mW1_ref, mb1_ref,
             mW2_ref, mb2_ref, pW1_ref, pb1_ref, pW2_ref, pb2_ref,
             pW3_ref, pb3_ref, out_ref):
        def leaky(t):
            return jnp.where(t >= 0, t, 0.2 * t)

        def elu(t):
            return jnp.where(t > 0, t, jnp.exp(jnp.minimum(t, 0.0)) - 1.0)

        xm = jnp.max(pmax_ref[...], axis=0)
        xsum = jnp.sum(psum_ref[...], axis=0)
        xmean = xsum / jnp.maximum(cnt_ref[...], 1.0)

        iaw = jnp.dot(attr_ref[...], iW1_ref[...],
                      preferred_element_type=jnp.float32) + ib1_ref[...]
        iaw = jnp.dot(leaky(iaw), iW2_ref[...],
                      preferred_element_type=jnp.float32) + ib2_ref[...]
        mu = jnp.mean(iaw, axis=1, keepdims=True)
        var = jnp.mean((iaw - mu) * (iaw - mu), axis=1, keepdims=True)
        iawn = (iaw - mu) * lax.rsqrt(var + 1e-5) * lng_ref[...] + lnb_ref[...]

        m = (jnp.dot(xm, mW1_ref[0:_DH, :],
                     preferred_element_type=jnp.float32)
             + jnp.dot(xmean, mW1_ref[_DH:2 * _DH, :],
                       preferred_element_type=jnp.float32)
             + jnp.dot(iawn, mW1_ref[2 * _DH:_DM, :],
                       preferred_element_type=jnp.float32)
             + mb1_ref[...])
        m = jnp.dot(leaky(m), mW2_ref[...],
                    preferred_element_type=jnp.float32) + mb2_ref[...]
        h2 = elu(jnp.dot(m, pW1_ref[...],
                         preferred_element_type=jnp.float32) + pb1_ref[...])
        h2 = elu(jnp.dot(h2, pW2_ref[...],
                         preferred_element_type=jnp.float32) + pb2_ref[...])
        out_ref[...] = jnp.dot(h2, pW3_ref[...],
                               preferred_element_type=jnp.float32) + pb3_ref[...]

    return pl.pallas_call(
        body,
        out_shape=jax.ShapeDtypeStruct((_NG, 1), jnp.float32),
    )(pmax, psum, cnt_f, iaw_attr, iW1, ib1, iW2, ib2, lng, lnb,
      mW1, mb1, mW2, mb2, pW1, pb1, pW2, pb2, pW3, pb3)


def kernel(x, edge_index, batch, iaw_attr, W0, b0, W1, b1, W2, b2,
           iaw_W1, iaw_b1, iaw_W2, iaw_b2, ln_g, ln_b,
           mg_W1, mg_b1, mg_W2, mg_b2, p_W1, p_b1, p_W2, p_b2, p_W3, p_b3):
    src = edge_index[0]
    dst = edge_index[1]

    dp = _build_sc_degree()(dst)
    dinv, cnt_f, batch16 = _tc_pre(dp[0].reshape(_N, 16),
                                   dp[1].reshape(_N, 16),
                                   batch.reshape(_N, 1))

    sc_conv = _build_sc_conv()
    src1 = jnp.concatenate([src, jnp.zeros((_EP - _E,), jnp.int32)])
    dst1 = jnp.concatenate([dst, jnp.full((_EP - _E,), _N, jnp.int32)])

    def conv(hws_chunks):
        outs = sc_conv(*hws_chunks, src1, dst1)
        return [o.reshape(_N, _CW) for o in outs]

    hws = _tc_mm_first(x, W0, dinv)
    s1 = conv(hws)
    hws2 = _tc_fused(s1, hws, dinv, b0.reshape(1, _DH), W1)
    s2 = conv(hws2)
    hws3 = _tc_fused(s2, hws2, dinv, b1.reshape(1, _DH), W2)
    s3 = conv(hws3)
    h3 = _tc_final(s3, hws3, dinv, b2.reshape(1, _DH))

    pmax, psum = _build_sc_pool()(h3, batch16)

    return _tc_head(pmax, psum, cnt_f, iaw_attr,
                    iaw_W1, iaw_b1.reshape(1, _DI),
                    iaw_W2, iaw_b2.reshape(1, _DI),
                    ln_g.reshape(1, _DI), ln_b.reshape(1, _DI),
                    mg_W1, mg_b1.reshape(1, _DM), mg_W2, mg_b2.reshape(1, _DM),
                    p_W1, p_b1.reshape(1, _DP), p_W2, p_b2.reshape(1, _DP),
                    p_W3, p_b3.reshape(1, 1))


# speedup vs baseline: 14.3025x; 1.0905x over previous
import functools

import jax
import jax.numpy as jnp
from jax import lax
from jax.experimental import pallas as pl
from jax.experimental.pallas import tpu as pltpu
from jax.experimental.pallas import tpu_sc as plsc

_N = 10000
_E = 320000
_NG = 64
_DF = 128
_DH = 512
_DI = 256
_DM = 2 * _DH + _DI
_DP = 512
_NC = 2
_NS = 16
_CW = 128
_K = 80
_RB = 400
_GRID = _N // _RB
_RPT = _N // _NS


@functools.lru_cache(maxsize=None)
def _mesh():
    return plsc.VectorSubcoreMesh(core_axis_name="c", subcore_axis_name="s",
                                  num_cores=_NC, num_subcores=_NS)


def _zero_vmem_2d(ref, rows, cols):
    z16 = jnp.zeros((16,), jnp.float32)
    nc = cols // 16

    def body(i, _):
        r = i // nc
        q = (i % nc) * 16
        ref[r, pl.ds(q, 16)] = z16
        return 0

    lax.fori_loop(0, rows * nc, body, 0)


@functools.lru_cache(maxsize=None)
def _build_sc_degree():
    @functools.partial(
        pl.kernel,
        out_type=jax.ShapeDtypeStruct((_NC, _NS, _RPT, 16), jnp.float32),
        mesh=_mesh(),
        scratch_types=[
            [pltpu.VMEM((_K,), jnp.int32) for _ in range(4)],
            pltpu.VMEM((_K, 16), jnp.float32),
            pltpu.VMEM((125, 16), jnp.float32),
            [pltpu.SemaphoreType.DMA for _ in range(4)],
            [pltpu.SemaphoreType.DMA for _ in range(4)],
            pltpu.VMEM_SHARED((_N, 16), jnp.float32),
        ],
    )
    def sc_degree(dst_hbm, out_hbm, idx, ones_v, zrow_v, isems, ssems, sdeg):
        c = lax.axis_index("c")
        s = lax.axis_index("s")
        wid = c * _NS + s

        one16 = jnp.full((16,), 1.0, jnp.float32)

        def fill(i, _):
            ones_v[i, :] = one16
            return 0

        lax.fori_loop(0, _K, fill, 0)
        _zero_vmem_2d(zrow_v, 125, 16)

        if True:
            def z(i, _):
                pltpu.sync_copy(zrow_v,
                                sdeg.at[pl.ds(s * _RPT + i * 125, 125)])
                return 0

            lax.fori_loop(0, _RPT // 125, z, 0)
            plsc.subcore_barrier()

            base = wid * (_E // (_NC * _NS))
            n = (_E // (_NC * _NS)) // _K

            def iload(i, sl):
                pltpu.async_copy(
                    dst_hbm.at[pl.ds(base + i * _K, _K)], idx[sl],
                    isems[sl])

            for p01 in range(2):
                iload(p01, p01)

            def step(t, _):
                for p in range(4):
                    i = t * 4 + p

                    @pl.when(i < n)
                    def _(i=i, p=p):
                        pltpu.make_async_copy(
                            dst_hbm.at[pl.ds(base + i * _K, _K)],
                            idx[p], isems[p]).wait()
                        pltpu.async_copy(
                            ones_v, sdeg.at[idx[p]], ssems[p], add=True)

                    @pl.when(jnp.logical_and(i >= 2, i - 2 < n))
                    def _(i=i, p=p):
                        pltpu.make_async_copy(
                            ones_v, sdeg.at[idx[(p + 2) % 4]],
                            ssems[(p + 2) % 4]).wait()

                    @pl.when(i + 2 < n)
                    def _(i=i, p=p):
                        iload(i + 2, (p + 2) % 4)

                return 0

            lax.fori_loop(0, (n + 3) // 4 + 1, step, 0)
            plsc.subcore_barrier()
            pltpu.sync_copy(
                sdeg.at[pl.ds(s * _RPT, _RPT)],
                out_hbm.at[c, s],
            )

    return sc_degree


_KC = 112
_NIT = 179
_EPT = _KC * _NIT
_EP = _NS * _EPT
_NPAD = _N + 8


@functools.lru_cache(maxsize=None)
def _build_sc_conv():
    @functools.partial(
        pl.kernel,
        out_type=[jax.ShapeDtypeStruct((_NS, _RPT, _CW), jnp.float32)] * 4,
        mesh=_mesh(),
        scratch_types=[
            [pltpu.VMEM((_KC,), jnp.int32) for _ in range(6)],
            [pltpu.VMEM((_KC,), jnp.int32) for _ in range(6)],
            [pltpu.VMEM((_KC, _CW), jnp.float32) for _ in range(3)],
            [pltpu.SemaphoreType.DMA for _ in range(6)],
            [pltpu.SemaphoreType.DMA for _ in range(3)],
            [pltpu.SemaphoreType.DMA for _ in range(3)],
            pltpu.VMEM_SHARED((_NPAD, _CW), jnp.float32),
        ],
    )
    def sc_conv(h0, h1, h2, h3, src_hbm, dst_hbm, zeros_hbm, o0, o1, o2, o3,
                sidx, didx, rbufs, isems, gsems, ssems, acc):
        c = lax.axis_index("c")
        s = lax.axis_index("s")
        ebase = s * _EPT
        n = _NIT

        def idx_load(j, sl):
            pltpu.async_copy(
                src_hbm.at[pl.ds(ebase + j * _KC, _KC)], sidx[sl],
                isems[sl])
            pltpu.async_copy(
                dst_hbm.at[pl.ds(ebase + j * _KC, _KC)], didx[sl],
                isems[sl])

        def idx_wait(j, sl):
            pltpu.make_async_copy(
                src_hbm.at[pl.ds(ebase + j * _KC, _KC)], sidx[sl],
                isems[sl]).wait()
            pltpu.make_async_copy(
                dst_hbm.at[pl.ds(ebase + j * _KC, _KC)], didx[sl],
                isems[sl]).wait()

        def one_chunk(hin, hout):
            pltpu.sync_copy(zeros_hbm, acc.at[pl.ds(s * _RPT, _RPT)])
            plsc.subcore_barrier()

            for p01 in range(2):
                idx_load(p01, p01)

            def step6(t, _):
                for p in range(6):
                    j = t * 6 + p

                    @pl.when(jnp.logical_and(j >= 3, j - 3 < n))
                    def _(j=j, p=p):
                        pltpu.make_async_copy(
                            rbufs[p % 3], acc.at[didx[(p + 3) % 6]],
                            ssems[p % 3]).wait()

                    @pl.when(j + 2 < n)
                    def _(j=j, p=p):
                        idx_load(j + 2, (p + 2) % 6)

                    @pl.when(j < n)
                    def _(j=j, p=p):
                        idx_wait(j, p)
                        pltpu.async_copy(
                            hin.at[sidx[p]], rbufs[p % 3],
                            gsems[p % 3])

                    jj = j - 2
                    qi = (p + 4) % 6
                    qr = (p + 1) % 3

                    @pl.when(jnp.logical_and(jj >= 0, jj < n))
                    def _(jj=jj, qi=qi, qr=qr):
                        pltpu.make_async_copy(
                            hin.at[sidx[qi]], rbufs[qr],
                            gsems[qr]).wait()
                        pltpu.async_copy(
                            rbufs[qr], acc.at[didx[qi]],
                            ssems[qr], add=True)

                return 0

            lax.fori_loop(0, (n + 2 + 6) // 6 + 1, step6, 0)
            plsc.subcore_barrier()
            pltpu.sync_copy(
                acc.at[pl.ds(s * _RPT, _RPT)],
                hout.at[s],
            )
            plsc.subcore_barrier()

        ins = [h0, h1, h2, h3]
        outs = [o0, o1, o2, o3]
        for ci in range(4):
            @pl.when(c == ci // 2)
            def _(ci=ci):
                one_chunk(ins[ci], outs[ci])

    return sc_conv


@functools.lru_cache(maxsize=None)
def _build_sc_pool():
    @functools.partial(
        pl.kernel,
        out_type=[
            jax.ShapeDtypeStruct((_NC * _NS, _NG, _DH), jnp.float32)
        ] * 2,
        mesh=_mesh(),
        scratch_types=[
            pltpu.VMEM((64, 16), jnp.int32),
            pltpu.VMEM((64, _DH), jnp.float32),
            pltpu.VMEM((1, _DH), jnp.float32),
            pltpu.VMEM((1, _DH), jnp.float32),
            pltpu.VMEM((16,), jnp.int32),
            pltpu.VMEM((_NG, _DH), jnp.float32),
            pltpu.VMEM((_NG, _DH), jnp.float32),
        ],
    )
    def sc_pool(h_hbm, b16_hbm, omax_hbm, osum_hbm,
                bbuf, rbuf, am, asm, pv, pmax, psum):
        cc = lax.axis_index("c")
        s = lax.axis_index("s")
        wid = cc * _NS + s

        r0 = (wid * _N // (_NC * _NS)) // 16 * 16
        r1 = ((wid + 1) * _N // (_NC * _NS)) // 16 * 16

        neg = jnp.full((16,), -3.4e38, jnp.float32)
        z16 = jnp.zeros((16,), jnp.float32)
        nch = _DH // 16

        def initrow(i, _):
            r = i // nch
            q = (i % nch) * 16
            pmax[r, pl.ds(q, 16)] = neg
            psum[r, pl.ds(q, 16)] = z16
            return 0

        lax.fori_loop(0, _NG * nch, initrow, 0)
        pv[...] = jnp.full((16,), -1, jnp.int32)

        nblk = (r1 - r0 + 63) // 64

        def blk(i, _):
            u = r0 + i * 64
            b = jnp.minimum(u, _N - 64)
            pltpu.sync_copy(h_hbm.at[pl.ds(b, 64)], rbuf)
            pltpu.sync_copy(b16_hbm.at[pl.ds(b, 64)], bbuf)

            def row(r, _):
                gr = b + r

                @pl.when(jnp.logical_and(gr >= u, gr < r1))
                def _():
                    b16 = bbuf[r, :]
                    sg = b16[0]
                    prev = pv[...][0]
                    eqs = sg == prev
                    pv[...] = b16

                    @pl.when(jnp.logical_and(jnp.logical_not(eqs),
                                             prev >= 0))
                    def _():
                        for f in range(nch):
                            pmax[prev, pl.ds(f * 16, 16)] = (
                                am[0, pl.ds(f * 16, 16)])
                            psum[prev, pl.ds(f * 16, 16)] = (
                                asm[0, pl.ds(f * 16, 16)])

                    for f in range(nch):
                        v = rbuf[r, pl.ds(f * 16, 16)]
                        m0 = am[0, pl.ds(f * 16, 16)]
                        s0 = asm[0, pl.ds(f * 16, 16)]
                        am[0, pl.ds(f * 16, 16)] = jnp.where(
                            eqs, jnp.maximum(m0, v), v)
                        asm[0, pl.ds(f * 16, 16)] = jnp.where(
                            eqs, s0 + v, v)

                return 0

            lax.fori_loop(0, 64, row, 0)
            return 0

        lax.fori_loop(0, nblk, blk, 0)
        last = pv[...][0]
        for f in range(nch):
            pmax[last, pl.ds(f * 16, 16)] = am[0, pl.ds(f * 16, 16)]
            psum[last, pl.ds(f * 16, 16)] = asm[0, pl.ds(f * 16, 16)]
        pltpu.sync_copy(pmax, omax_hbm.at[wid])
        pltpu.sync_copy(psum, osum_hbm.at[wid])

    return sc_pool


def _tc_pre(dp0, dp1, batch_col):
    def body(dp0_ref, dp1_ref, b_ref, dinv_ref, cf_ref, b16_ref, acc_ref):
        i = pl.program_id(0)
        deg = dp0_ref[:, 0:1] + dp1_ref[:, 0:1] + 1.0
        dinv_ref[...] = lax.rsqrt(deg)
        b16_ref[...] = jnp.broadcast_to(b_ref[...], (_RB, 16))

        oh = (lax.broadcasted_iota(jnp.int32, (_RB, _NG), 1)
              == jnp.broadcast_to(b_ref[...], (_RB, _NG))).astype(jnp.float32)
        part = jnp.dot(jnp.ones((1, _RB), jnp.float32), oh,
                       preferred_element_type=jnp.float32)

        @pl.when(i == 0)
        def _():
            acc_ref[...] = jnp.zeros((1, _NG), jnp.float32)

        acc_ref[...] += part

        @pl.when(i == _GRID - 1)
        def _():
            cr = acc_ref[...]
            i0 = lax.broadcasted_iota(jnp.int32, (_NG, _NG), 0)
            i1 = lax.broadcasted_iota(jnp.int32, (_NG, _NG), 1)
            eye = (i0 == i1).astype(jnp.float32)
            ones = jnp.ones((_NG, 1), jnp.float32)
            cf_ref[...] = jnp.dot(eye * cr, ones,
                                  preferred_element_type=jnp.float32)

    return pl.pallas_call(
        body,
        grid=(_GRID,),
        in_specs=[
            pl.BlockSpec((_RB, 16), lambda i: (i, 0)),
            pl.BlockSpec((_RB, 16), lambda i: (i, 0)),
            pl.BlockSpec((_RB, 1), lambda i: (i, 0)),
        ],
        scratch_shapes=[pltpu.VMEM((1, _NG), jnp.float32)],
        out_specs=[
            pl.BlockSpec((_RB, 1), lambda i: (i, 0)),
            pl.BlockSpec((_NG, 1), lambda i: (0, 0)),
            pl.BlockSpec((_RB, 16), lambda i: (i, 0)),
        ],
        out_shape=[
            jax.ShapeDtypeStruct((_N, 1), jnp.float32),
            jax.ShapeDtypeStruct((_NG, 1), jnp.float32),
            jax.ShapeDtypeStruct((_N, 16), jnp.int32),
        ],
    )(dp0, dp1, batch_col)


def _tc_mm_first(x, w, dinv):
    def body(x_ref, w_ref, d_ref, o0, o1, o2, o3):
        h = jnp.dot(x_ref[...], w_ref[...],
                    preferred_element_type=jnp.float32) * d_ref[...]
        for ci, o in enumerate((o0, o1, o2, o3)):
            o[...] = h[:, ci * _CW:(ci + 1) * _CW]

    return pl.pallas_call(
        body,
        grid=(_GRID,),
        in_specs=[
            pl.BlockSpec((_RB, _DF), lambda i: (i, 0)),
            pl.BlockSpec((_DF, _DH), lambda i: (0, 0)),
            pl.BlockSpec((_RB, 1), lambda i: (i, 0)),
        ],
        out_specs=[pl.BlockSpec((_RB, _CW), lambda i: (i, 0))] * 4,
        out_shape=[jax.ShapeDtypeStruct((_N, _CW), jnp.float32)] * 4,
    )(x, w, dinv)


def _tc_fused(sc, hws, dinv, b, w):
    def body(s0, s1, s2, s3, p0, p1, p2, p3, d_ref, b_ref, w_ref,
             o0, o1, o2, o3):
        d = d_ref[...]
        hs = []
        for ci, (sr, pr) in enumerate(zip((s0, s1, s2, s3),
                                          (p0, p1, p2, p3))):
            hs.append((sr[...] + pr[...]) * d
                      + b_ref[:, ci * _CW:(ci + 1) * _CW])
        h = jnp.concatenate(hs, axis=1)
        g = jnp.dot(h, w_ref[...], preferred_element_type=jnp.float32) * d
        for ci, o in enumerate((o0, o1, o2, o3)):
            o[...] = g[:, ci * _CW:(ci + 1) * _CW]

    return pl.pallas_call(
        body,
        grid=(_GRID,),
        in_specs=(
            [pl.BlockSpec((_RB, _CW), lambda i: (i, 0))] * 8
            + [pl.BlockSpec((_RB, 1), lambda i: (i, 0)),
               pl.BlockSpec((1, _DH), lambda i: (0, 0)),
               pl.BlockSpec((_DH, _DH), lambda i: (0, 0))]
        ),
        out_specs=[pl.BlockSpec((_RB, _CW), lambda i: (i, 0))] * 4,
        out_shape=[jax.ShapeDtypeStruct((_N, _CW), jnp.float32)] * 4,
    )(*sc, *hws, dinv, b, w)


def _tc_final(sc, hws, dinv, b):
    def body(s0, s1, s2, s3, p0, p1, p2, p3, d_ref, b_ref, o_ref):
        d = d_ref[...]
        hs = []
        for ci, (sr, pr) in enumerate(zip((s0, s1, s2, s3),
                                          (p0, p1, p2, p3))):
            hs.append((sr[...] + pr[...]) * d
                      + b_ref[:, ci * _CW:(ci + 1) * _CW])
        o_ref[...] = jnp.concatenate(hs, axis=1)

    return pl.pallas_call(
        body,
        grid=(_GRID,),
        in_specs=(
            [pl.BlockSpec((_RB, _CW), lambda i: (i, 0))] * 8
            + [pl.BlockSpec((_RB, 1), lambda i: (i, 0)),
               pl.BlockSpec((1, _DH), lambda i: (0, 0))]
        ),
        out_specs=pl.BlockSpec((_RB, _DH), lambda i: (i, 0)),
        out_shape=jax.ShapeDtypeStruct((_N, _DH), jnp.float32),
    )(*sc, *hws, dinv, b)


def _tc_head(pmax, psum, cnt_f, iaw_attr, iW1, ib1, iW2, ib2, lng, lnb,
             mW1, mb1, mW2, mb2, pW1, pb1, pW2, pb2, pW3, pb3):
    def body(pmax_ref, psum_ref, cnt_ref, attr_ref, iW1_ref, ib1_ref,
             iW2_ref, ib2_ref, lng_ref, lnb_ref, mW1_ref, mb1_ref,
             mW2_ref, mb2_ref, pW1_ref, pb1_ref, pW2_ref, pb2_ref,
             pW3_ref, pb3_ref, out_ref):
        def leaky(t):
            return jnp.where(t >= 0, t, 0.2 * t)

        def elu(t):
            return jnp.where(t > 0, t, jnp.exp(jnp.minimum(t, 0.0)) - 1.0)

        xm = jnp.max(pmax_ref[...], axis=0)
        xsum = jnp.sum(psum_ref[...], axis=0)
        xmean = xsum / jnp.maximum(cnt_ref[...], 1.0)

        iaw = jnp.dot(attr_ref[...], iW1_ref[...],
                      preferred_element_type=jnp.float32) + ib1_ref[...]
        iaw = jnp.dot(leaky(iaw), iW2_ref[...],
                      preferred_element_type=jnp.float32) + ib2_ref[...]
        mu = jnp.mean(iaw, axis=1, keepdims=True)
        var = jnp.mean((iaw - mu) * (iaw - mu), axis=1, keepdims=True)
        iawn = (iaw - mu) * lax.rsqrt(var + 1e-5) * lng_ref[...] + lnb_ref[...]

        m = (jnp.dot(xm, mW1_ref[0:_DH, :],
                     preferred_element_type=jnp.float32)
             + jnp.dot(xmean, mW1_ref[_DH:2 * _DH, :],
                       preferred_element_type=jnp.float32)
             + jnp.dot(iawn, mW1_ref[2 * _DH:_DM, :],
                       preferred_element_type=jnp.float32)
             + mb1_ref[...])
        m = jnp.dot(leaky(m), mW2_ref[...],
                    preferred_element_type=jnp.float32) + mb2_ref[...]
        h2 = elu(jnp.dot(m, pW1_ref[...],
                         preferred_element_type=jnp.float32) + pb1_ref[...])
        h2 = elu(jnp.dot(h2, pW2_ref[...],
                         preferred_element_type=jnp.float32) + pb2_ref[...])
        out_ref[...] = jnp.dot(h2, pW3_ref[...],
                               preferred_element_type=jnp.float32) + pb3_ref[...]

    return pl.pallas_call(
        body,
        out_shape=jax.ShapeDtypeStruct((_NG, 1), jnp.float32),
    )(pmax, psum, cnt_f, iaw_attr, iW1, ib1, iW2, ib2, lng, lnb,
      mW1, mb1, mW2, mb2, pW1, pb1, pW2, pb2, pW3, pb3)


def kernel(x, edge_index, batch, iaw_attr, W0, b0, W1, b1, W2, b2,
           iaw_W1, iaw_b1, iaw_W2, iaw_b2, ln_g, ln_b,
           mg_W1, mg_b1, mg_W2, mg_b2, p_W1, p_b1, p_W2, p_b2, p_W3, p_b3):
    src = edge_index[0]
    dst = edge_index[1]

    dp = _build_sc_degree()(dst)
    dinv, cnt_f, batch16 = _tc_pre(dp[0].reshape(_N, 16),
                                   dp[1].reshape(_N, 16),
                                   batch.reshape(_N, 1))

    sc_conv = _build_sc_conv()
    src1 = jnp.concatenate([src, jnp.zeros((_EP - _E,), jnp.int32)])
    dst1 = jnp.concatenate([dst, jnp.full((_EP - _E,), _N, jnp.int32)])

    zrows = jnp.zeros((_RPT, _CW), jnp.float32)

    def conv(hws_chunks):
        outs = sc_conv(*hws_chunks, src1, dst1, zrows)
        return [o.reshape(_N, _CW) for o in outs]

    hws = _tc_mm_first(x, W0, dinv)
    s1 = conv(hws)
    hws2 = _tc_fused(s1, hws, dinv, b0.reshape(1, _DH), W1)
    s2 = conv(hws2)
    hws3 = _tc_fused(s2, hws2, dinv, b1.reshape(1, _DH), W2)
    s3 = conv(hws3)
    h3 = _tc_final(s3, hws3, dinv, b2.reshape(1, _DH))

    pmax, psum = _build_sc_pool()(h3, batch16)

    return _tc_head(pmax, psum, cnt_f, iaw_attr,
                    iaw_W1, iaw_b1.reshape(1, _DI),
                    iaw_W2, iaw_b2.reshape(1, _DI),
                    ln_g.reshape(1, _DI), ln_b.reshape(1, _DI),
                    mg_W1, mg_b1.reshape(1, _DM), mg_W2, mg_b2.reshape(1, _DM),
                    p_W1, p_b1.reshape(1, _DP), p_W2, p_b2.reshape(1, _DP),
                    p_W3, p_b3.reshape(1, 1))
